# Initial kernel scaffold; baseline (speedup 1.0000x reference)
import functools

import jax
import jax.numpy as jnp
from jax import lax
from jax.experimental import pallas as pl
from jax.experimental.pallas import tpu as pltpu
from jax.experimental.pallas import tpu_sc as plsc

IMAGE_TOKEN = 32000
_NUM_CORES = 2
_NUM_SUBCORES = 16
_NW = _NUM_CORES * _NUM_SUBCORES
_BLK = 8


def _mask_pos_kernel(ids_ref, mask_ref, outmask_ref, pos_ref, pvec_ref, *, S, P, E):
    B = ids_ref.shape[0]
    lane_e = lax.broadcasted_iota(jnp.int32, (1, E), 1)
    lane_s = lax.broadcasted_iota(jnp.int32, (1, S), 1)
    lane16 = lax.broadcasted_iota(jnp.int32, (1, 16), 1)
    pvec = jnp.zeros((1, 16), jnp.int32)
    zeros_shift = jnp.zeros((1, P - 1), jnp.int32)
    for b in range(B):
        ids = ids_ref[b:b + 1, :]
        m = mask_ref[b:b + 1, :]
        p = jnp.sum(jnp.where(ids == IMAGE_TOKEN, lane_s, 0))
        a_low = jnp.concatenate([m, zeros_shift], axis=1)
        a_high = jnp.concatenate([zeros_shift, m], axis=1)
        sel = jnp.where(lane_e < p, a_low,
                        jnp.where(lane_e < p + P, jnp.int32(1), a_high))
        cs = sel
        sh = 1
        while sh < E:
            cs = cs + jnp.concatenate(
                [jnp.zeros((1, sh), jnp.int32), cs[:, :E - sh]], axis=1)
            sh *= 2
        pos = cs - 1
        pos = jnp.where(sel == 0, 1, pos)
        outmask_ref[b:b + 1, :] = sel
        pos_ref[b:b + 1, :] = pos
        pvec = jnp.where(lane16 == b, p, pvec)
    pvec_ref[...] = pvec


def _row_copy(emb_hbm, img_hbm, out_hbm, buf, b, r, p, P):
    row = buf.at[pl.ds(0, 1)]
    dst = out_hbm.at[b, pl.ds(r, 1), :]

    @pl.when(r < p)
    def _():
        pltpu.sync_copy(emb_hbm.at[b, pl.ds(r, 1), :], row)
        pltpu.sync_copy(row, dst)

    @pl.when((r >= p) & (r < p + P))
    def _():
        pltpu.sync_copy(img_hbm.at[b, pl.ds(r - p, 1), :], row)
        pltpu.sync_copy(row, dst)

    @pl.when(r >= p + P)
    def _():
        pltpu.sync_copy(emb_hbm.at[b, pl.ds(r - (P - 1), 1), :], row)
        pltpu.sync_copy(row, dst)


def _sc_copy_kernel(emb_hbm, img_hbm, pvec_hbm, out_hbm, buf, pbuf, *, B, S, P, E):
    cid = lax.axis_index("core")
    sid = lax.axis_index("subcore")
    wid = cid * _NUM_SUBCORES + sid
    pltpu.sync_copy(pvec_hbm, pbuf)
    NB = E // _BLK
    TAIL0 = NB * _BLK
    KMAX = (NB + _NW - 1) // _NW
    for b in range(B):
        p = pbuf[0, b]
        hi0 = p + P
        for k in range(KMAX):
            blk = k * _NW + wid

            @pl.when(blk < NB)
            def _(blk=blk, p=p, hi0=hi0, b=b):
                r0 = blk * _BLK

                @pl.when(r0 + _BLK <= p)
                def _():
                    pltpu.sync_copy(emb_hbm.at[b, pl.ds(r0, _BLK), :], buf)
                    pltpu.sync_copy(buf, out_hbm.at[b, pl.ds(r0, _BLK), :])

                @pl.when((r0 >= p) & (r0 + _BLK <= hi0))
                def _():
                    pltpu.sync_copy(img_hbm.at[b, pl.ds(r0 - p, _BLK), :], buf)
                    pltpu.sync_copy(buf, out_hbm.at[b, pl.ds(r0, _BLK), :])

                @pl.when(r0 >= hi0)
                def _():
                    pltpu.sync_copy(emb_hbm.at[b, pl.ds(r0 - (P - 1), _BLK), :], buf)
                    pltpu.sync_copy(buf, out_hbm.at[b, pl.ds(r0, _BLK), :])

                straddle = ((r0 < p) & (r0 + _BLK > p)) | \
                           ((r0 < hi0) & (r0 + _BLK > hi0))

                @pl.when(straddle)
                def _():
                    @pl.loop(r0, r0 + _BLK)
                    def _(r):
                        _row_copy(emb_hbm, img_hbm, out_hbm, buf, b, r, p, P)

        @pl.when(wid == b)
        def _(p=p, b=b):
            @pl.loop(TAIL0, E)
            def _(r):
                _row_copy(emb_hbm, img_hbm, out_hbm, buf, b, r, p, P)


def kernel(inputs_embeds, image_features, input_ids, attention_mask):
    B, S, D = inputs_embeds.shape
    P = image_features.shape[1]
    E = S + P - 1

    i32 = jnp.int32
    outmask, pos, pvec = pl.pallas_call(
        functools.partial(_mask_pos_kernel, S=S, P=P, E=E),
        out_shape=[
            jax.ShapeDtypeStruct((B, E), i32),
            jax.ShapeDtypeStruct((B, E), i32),
            jax.ShapeDtypeStruct((1, 16), i32),
        ],
    )(input_ids.astype(i32), attention_mask.astype(i32))

    mesh = plsc.VectorSubcoreMesh(core_axis_name="core",
                                  subcore_axis_name="subcore")
    sc_fn = pl.kernel(
        functools.partial(_sc_copy_kernel, B=B, S=S, P=P, E=E),
        out_type=jax.ShapeDtypeStruct((B, E, D), inputs_embeds.dtype),
        mesh=mesh,
        scratch_types=[
            pltpu.VMEM((_BLK, D), inputs_embeds.dtype),
            pltpu.VMEM((1, 16), i32),
        ],
    )
    final = sc_fn(inputs_embeds, image_features, pvec)
    return final, outmask.astype(attention_mask.dtype), pos


# speedup vs baseline: 1.1250x; 1.1250x over previous
import functools

import jax
import jax.numpy as jnp
from jax import lax
from jax.experimental import pallas as pl
from jax.experimental.pallas import tpu as pltpu
from jax.experimental.pallas import tpu_sc as plsc

IMAGE_TOKEN = 32000
_NUM_CORES = 2
_NUM_SUBCORES = 16
_NW = _NUM_CORES * _NUM_SUBCORES
_BLK = 8


def _mask_pos_kernel(ids_ref, mask_ref, outmask_ref, pos_ref, pvec_ref, *, S, P, E):
    B = ids_ref.shape[0]
    lane_e = lax.broadcasted_iota(jnp.int32, (1, E), 1)
    lane_s = lax.broadcasted_iota(jnp.int32, (1, S), 1)
    lane16 = lax.broadcasted_iota(jnp.int32, (1, 16), 1)
    pvec = jnp.zeros((1, 16), jnp.int32)
    zeros_shift = jnp.zeros((1, P - 1), jnp.int32)
    for b in range(B):
        ids = ids_ref[b:b + 1, :]
        m = mask_ref[b:b + 1, :]
        p = jnp.sum(jnp.where(ids == IMAGE_TOKEN, lane_s, 0))
        a_low = jnp.concatenate([m, zeros_shift], axis=1)
        a_high = jnp.concatenate([zeros_shift, m], axis=1)
        sel = jnp.where(lane_e < p, a_low,
                        jnp.where(lane_e < p + P, jnp.int32(1), a_high))
        cs = sel
        sh = 1
        while sh < E:
            cs = cs + jnp.concatenate(
                [jnp.zeros((1, sh), jnp.int32), cs[:, :E - sh]], axis=1)
            sh *= 2
        pos = cs - 1
        pos = jnp.where(sel == 0, 1, pos)
        outmask_ref[b:b + 1, :] = sel
        pos_ref[b:b + 1, :] = pos
        pvec = jnp.where(lane16 == b, p, pvec)
    pvec_ref[...] = pvec


def _row_copy(emb_hbm, img_hbm, out_hbm, buf, b, r, p, *, S, P, E, D):
    row = buf.at[pl.ds(0, D)]
    dst = out_hbm.at[pl.ds((b * E + r) * D, D)]

    @pl.when(r < p)
    def _():
        pltpu.sync_copy(emb_hbm.at[pl.ds((b * S + r) * D, D)], row)
        pltpu.sync_copy(row, dst)

    @pl.when((r >= p) & (r < p + P))
    def _():
        pltpu.sync_copy(img_hbm.at[pl.ds((b * P + r - p) * D, D)], row)
        pltpu.sync_copy(row, dst)

    @pl.when(r >= p + P)
    def _():
        pltpu.sync_copy(emb_hbm.at[pl.ds((b * S + r - (P - 1)) * D, D)], row)
        pltpu.sync_copy(row, dst)


def _sc_copy_kernel(emb_hbm, img_hbm, pvec_hbm, out_hbm, buf, pbuf, *, B, S, P, E, D):
    cid = lax.axis_index("core")
    sid = lax.axis_index("subcore")
    wid = cid * _NUM_SUBCORES + sid
    pltpu.sync_copy(pvec_hbm, pbuf)
    pvals = pbuf[...]
    NB = E // _BLK
    TAIL0 = NB * _BLK
    KMAX = (NB + _NW - 1) // _NW
    W = _BLK * D
    for b in range(B):
        p = pvals[b]
        hi0 = p + P
        for k in range(KMAX):
            blk = k * _NW + wid

            @pl.when(blk < NB)
            def _(blk=blk, p=p, hi0=hi0, b=b):
                r0 = blk * _BLK
                dst = out_hbm.at[pl.ds((b * E + r0) * D, W)]

                @pl.when(r0 + _BLK <= p)
                def _():
                    pltpu.sync_copy(emb_hbm.at[pl.ds((b * S + r0) * D, W)], buf)
                    pltpu.sync_copy(buf, dst)

                @pl.when((r0 >= p) & (r0 + _BLK <= hi0))
                def _():
                    pltpu.sync_copy(img_hbm.at[pl.ds((b * P + r0 - p) * D, W)], buf)
                    pltpu.sync_copy(buf, dst)

                @pl.when(r0 >= hi0)
                def _():
                    pltpu.sync_copy(
                        emb_hbm.at[pl.ds((b * S + r0 - (P - 1)) * D, W)], buf)
                    pltpu.sync_copy(buf, dst)

                straddle = ((r0 < p) & (r0 + _BLK > p)) | \
                           ((r0 < hi0) & (r0 + _BLK > hi0))

                @pl.when(straddle)
                def _():
                    @pl.loop(r0, r0 + _BLK)
                    def _(r):
                        _row_copy(emb_hbm, img_hbm, out_hbm, buf, b, r, p,
                                  S=S, P=P, E=E, D=D)

        @pl.when(wid == b)
        def _(p=p, b=b):
            @pl.loop(TAIL0, E)
            def _(r):
                _row_copy(emb_hbm, img_hbm, out_hbm, buf, b, r, p,
                          S=S, P=P, E=E, D=D)


def kernel(inputs_embeds, image_features, input_ids, attention_mask):
    B, S, D = inputs_embeds.shape
    P = image_features.shape[1]
    E = S + P - 1

    i32 = jnp.int32
    outmask, pos, pvec = pl.pallas_call(
        functools.partial(_mask_pos_kernel, S=S, P=P, E=E),
        out_shape=[
            jax.ShapeDtypeStruct((B, E), i32),
            jax.ShapeDtypeStruct((B, E), i32),
            jax.ShapeDtypeStruct((1, 16), i32),
        ],
    )(input_ids.astype(i32), attention_mask.astype(i32))

    mesh = plsc.VectorSubcoreMesh(core_axis_name="core",
                                  subcore_axis_name="subcore")
    sc_fn = pl.kernel(
        functools.partial(_sc_copy_kernel, B=B, S=S, P=P, E=E, D=D),
        out_type=jax.ShapeDtypeStruct((B * E * D,), inputs_embeds.dtype),
        mesh=mesh,
        scratch_types=[
            pltpu.VMEM((_BLK * D,), inputs_embeds.dtype),
            pltpu.VMEM((16,), i32),
        ],
    )
    final = sc_fn(inputs_embeds.reshape(-1), image_features.reshape(-1),
                  pvec.reshape(-1))
    return (final.reshape(B, E, D), outmask.astype(attention_mask.dtype), pos)
